# Initial kernel scaffold; baseline (speedup 1.0000x reference)
import jax
import jax.numpy as jnp
from jax.experimental import pallas as pl


def kernel(x_driver, x_rider, edge_serves, edge_served_by, W1_serves, b1_serves, W1_served_by, b1_served_by, W2_serves, b2_serves, W2_served_by, b2_served_by, W3_serves, b3_serves, W3_served_by, b3_served_by):
    raise NotImplementedError("write your pallas kernel here")


# speedup vs baseline: 3.1240x; 3.1240x over previous
import functools

import jax
import jax.numpy as jnp
from jax import lax
from jax.experimental import pallas as pl
from jax.experimental.pallas import tpu as pltpu
from jax.experimental.pallas import tpu_sc as plsc

NN = 10000
EE = 160000
DIN = 256
HID = 256
DOUT = 128
QW = 64

NC, NS = 2, 16
CH = 128
EP = 163840
CH16 = EP // (NS * CH)
NP = 10240
RB = NP // NS
BR = 400
GRID = NN // BR

f32 = jnp.float32
i32 = jnp.int32

_MESH = plsc.VectorSubcoreMesh(core_axis_name="c", subcore_axis_name="s",
                               num_cores=NC, num_subcores=NS)


def _hist_body(hidx, ones_h, z16, out, idx_v, ones_v, z_v, acc_a, acc_b):
    cid = lax.axis_index("c")
    tid = lax.axis_index("s")
    pltpu.sync_copy(ones_h, ones_v)
    pltpu.sync_copy(z16, z_v)
    pltpu.sync_copy(z_v, acc_a.at[pl.ds(tid * RB, RB)])
    pltpu.sync_copy(z_v, acc_b.at[pl.ds(tid * RB, RB)])
    plsc.subcore_barrier()
    pltpu.sync_copy(hidx.at[(2 * cid) * NS + tid], idx_v)

    @pl.loop(0, CH16)
    def _(j):
        pltpu.sync_copy(ones_v, acc_a.at[idx_v.at[j]], add=True)

    pltpu.sync_copy(hidx.at[(2 * cid + 1) * NS + tid], idx_v)

    @pl.loop(0, CH16)
    def _(j):
        pltpu.sync_copy(ones_v, acc_b.at[idx_v.at[j]], add=True)

    plsc.subcore_barrier()
    pltpu.sync_copy(acc_a.at[pl.ds(tid * RB, RB)],
                    out.at[pl.ds((2 * cid) * NP + tid * RB, RB)])
    pltpu.sync_copy(acc_b.at[pl.ds(tid * RB, RB)],
                    out.at[pl.ds((2 * cid + 1) * NP + tid * RB, RB)])


_hist = pl.kernel(
    _hist_body,
    out_type=jax.ShapeDtypeStruct((4 * NP, 16), f32),
    mesh=_MESH,
    compiler_params=pltpu.CompilerParams(use_tc_tiling_on_sc=False),
    scratch_types=[
        pltpu.VMEM((CH16, CH), i32),
        pltpu.VMEM((CH, 16), f32),
        pltpu.VMEM((RB, 16), f32),
        pltpu.VMEM_SHARED((NP, 16), f32),
        pltpu.VMEM_SHARED((NP, 16), f32),
    ],
)


def _gs_body(npass, m_flat, isrc, idst, zz, out,
             isrc_v, idst_v, rows_v, z_v, acc):
    cid = lax.axis_index("c")
    tid = lax.axis_index("s")
    pltpu.sync_copy(idst.at[tid], idst_v)
    pltpu.sync_copy(zz, z_v)
    for k in range(npass):
        pltpu.sync_copy(isrc.at[(k * NC + cid) * NS + tid], isrc_v)
        for r in range(RB // CH):
            pltpu.sync_copy(z_v, acc.at[pl.ds(tid * RB + r * CH, CH)])
        plsc.subcore_barrier()

        @pl.loop(0, CH16)
        def _(j):
            pltpu.sync_copy(m_flat.at[isrc_v.at[j]], rows_v)
            pltpu.sync_copy(rows_v, acc.at[idst_v.at[j]], add=True)

        plsc.subcore_barrier()
        pltpu.sync_copy(acc.at[pl.ds(tid * RB, RB)],
                        out.at[pl.ds((k * NC + cid) * NP + tid * RB, RB)])
        if k + 1 < npass:
            plsc.subcore_barrier()


def _mk_gs(npass):
    return pl.kernel(
        functools.partial(_gs_body, npass),
        out_type=jax.ShapeDtypeStruct((npass * NC * NP, QW), f32),
        mesh=_MESH,
        compiler_params=pltpu.CompilerParams(use_tc_tiling_on_sc=False),
        scratch_types=[
            pltpu.VMEM((CH16, CH), i32),
            pltpu.VMEM((CH16, CH), i32),
            pltpu.VMEM((CH, QW), f32),
            pltpu.VMEM((CH, QW), f32),
            pltpu.VMEM_SHARED((NP, QW), f32),
        ],
    )


_gs2 = _mk_gs(2)
_gs1 = _mk_gs(1)


def _rs(ref):
    return lax.rsqrt(jnp.maximum(ref[0, :, 0:1], 1.0))


def _quarters(m, n):
    return jnp.stack([m[:, QW * q:QW * (q + 1)] for q in range(n)])


def _tc1_body(x_ref, d_ref, o_ref):
    m = x_ref[...] * _rs(d_ref)
    o_ref[...] = _quarters(m, 4)


def _tc2_body(a_ref, di_ref, do_ref, w_ref, b_ref, o_ref):
    si = _rs(di_ref)
    h = b_ref[...]
    for q in range(4):
        h = h + jnp.dot(a_ref[q] * si, w_ref[QW * q:QW * (q + 1)],
                        preferred_element_type=f32)
    m = jnp.maximum(h, 0.0) * _rs(do_ref)
    o_ref[...] = _quarters(m, 4)


def _tc3_body(a_ref, di_ref, do_ref, w2_ref, b2_ref, w3_ref, o_ref):
    si = _rs(di_ref)
    h = b2_ref[...]
    for q in range(4):
        h = h + jnp.dot(a_ref[q] * si, w2_ref[QW * q:QW * (q + 1)],
                        preferred_element_type=f32)
    h = jnp.maximum(h, 0.0) * _rs(do_ref)
    t = jnp.dot(h, w3_ref[...], preferred_element_type=f32)
    o_ref[...] = _quarters(t, 2)


def _tc4_body(p_ref, di_ref, b_ref, o_ref):
    si = _rs(di_ref)
    o_ref[...] = jnp.concatenate([p_ref[0], p_ref[1]], axis=1) * si + b_ref[...]


def _deg_spec(h):
    return pl.BlockSpec((1, BR, 16), lambda i, h=h: (h, i, 0))


_tc1 = pl.pallas_call(
    _tc1_body,
    grid=(GRID,),
    in_specs=[pl.BlockSpec((BR, DIN), lambda i: (i, 0)), _deg_spec(0)],
    out_specs=pl.BlockSpec((4, BR, QW), lambda i: (0, i, 0)),
    out_shape=jax.ShapeDtypeStruct((4, NN, QW), f32),
)

_tc2 = pl.pallas_call(
    _tc2_body,
    grid=(GRID,),
    in_specs=[
        pl.BlockSpec((4, BR, QW), lambda i: (0, i, 0)),
        _deg_spec(1),
        _deg_spec(2),
        pl.BlockSpec((HID, HID), lambda i: (0, 0)),
        pl.BlockSpec((1, HID), lambda i: (0, 0)),
    ],
    out_specs=pl.BlockSpec((4, BR, QW), lambda i: (0, i, 0)),
    out_shape=jax.ShapeDtypeStruct((4, NN, QW), f32),
)

_tc3 = pl.pallas_call(
    _tc3_body,
    grid=(GRID,),
    in_specs=[
        pl.BlockSpec((4, BR, QW), lambda i: (0, i, 0)),
        _deg_spec(3),
        _deg_spec(0),
        pl.BlockSpec((HID, HID), lambda i: (0, 0)),
        pl.BlockSpec((1, HID), lambda i: (0, 0)),
        pl.BlockSpec((HID, DOUT), lambda i: (0, 0)),
    ],
    out_specs=pl.BlockSpec((2, BR, QW), lambda i: (0, i, 0)),
    out_shape=jax.ShapeDtypeStruct((2, NN, QW), f32),
)

_tc4 = pl.pallas_call(
    _tc4_body,
    grid=(GRID,),
    in_specs=[
        pl.BlockSpec((2, BR, QW), lambda i: (0, i, 0)),
        _deg_spec(1),
        pl.BlockSpec((1, DOUT), lambda i: (0, 0)),
    ],
    out_specs=pl.BlockSpec((BR, DOUT), lambda i: (i, 0)),
    out_shape=jax.ShapeDtypeStruct((NN, DOUT), f32),
)


def kernel(x_driver, x_rider, edge_serves, edge_served_by,
           W1_serves, b1_serves, W1_served_by, b1_served_by,
           W2_serves, b2_serves, W2_served_by, b2_served_by,
           W3_serves, b3_serves, W3_served_by, b3_served_by):
    s_src, s_dst = edge_serves[0], edge_serves[1]
    sb_src, sb_dst = edge_served_by[0], edge_served_by[1]
    pad0 = jnp.zeros((EP - EE,), i32)
    padn = jnp.full((EP - EE,), NN, i32)

    def cat(a, p):
        return jnp.concatenate([a, p])

    hidx = jnp.stack([cat(s_src, padn), cat(s_dst, padn),
                      cat(sb_src, padn), cat(sb_dst, padn)])
    hidx = hidx.reshape(4 * NS, CH16, CH)

    g1s = cat(s_src, pad0)
    g1src = jnp.stack([g1s + q * NN for q in range(4)]).reshape(
        4 * NS, CH16, CH)
    g1dst = cat(s_dst, padn).reshape(NS, CH16, CH)
    g2s = cat(sb_src, pad0)
    g2src = jnp.stack([g2s + q * NN for q in range(4)]).reshape(
        4 * NS, CH16, CH)
    g2dst = cat(sb_dst, padn).reshape(NS, CH16, CH)
    g3src = jnp.stack([g1s, g1s + NN]).reshape(2 * NS, CH16, CH)
    g3dst = g1dst

    ones16 = jnp.ones((CH, 16), f32)
    z16 = jnp.zeros((RB, 16), f32)
    zq = jnp.zeros((CH, QW), f32)

    degs = _hist(hidx, ones16, z16).reshape(4, NP, 16)

    m1 = _tc1(x_rider, degs)
    agg1 = _gs2(m1.reshape(4 * NN, QW), g1src, g1dst, zq)
    m2 = _tc2(agg1.reshape(4, NP, QW), degs, degs, W1_serves,
              b1_serves.reshape(1, HID))
    agg2 = _gs2(m2.reshape(4 * NN, QW), g2src, g2dst, zq)
    t3 = _tc3(agg2.reshape(4, NP, QW), degs, degs, W2_served_by,
              b2_served_by.reshape(1, HID), W3_serves)
    p3 = _gs1(t3.reshape(2 * NN, QW), g3src, g3dst, zq)
    return _tc4(p3.reshape(2, NP, QW), degs,
                b3_serves.reshape(1, DOUT))


# speedup vs baseline: 3.4222x; 1.0955x over previous
import functools

import jax
import jax.numpy as jnp
from jax import lax
from jax.experimental import pallas as pl
from jax.experimental.pallas import tpu as pltpu
from jax.experimental.pallas import tpu_sc as plsc

NN = 10000
EE = 160000
DIN = 256
HID = 256
DOUT = 128
QW = 64

NC, NS = 2, 16
CH = 256
EP = 163840
CH16 = EP // (NS * CH)
NP = 10240
RB = NP // NS
ZCH = 128
BR = 400
GRID = NN // BR

f32 = jnp.float32
i32 = jnp.int32

_MESH = plsc.VectorSubcoreMesh(core_axis_name="c", subcore_axis_name="s",
                               num_cores=NC, num_subcores=NS)


def _hist_body(hidx, ones_h, z16, out, idx_v, ones_v, z_v, acc_a, acc_b):
    cid = lax.axis_index("c")
    tid = lax.axis_index("s")
    pltpu.sync_copy(ones_h, ones_v)
    pltpu.sync_copy(z16, z_v)
    pltpu.sync_copy(z_v, acc_a.at[pl.ds(tid * RB, RB)])
    pltpu.sync_copy(z_v, acc_b.at[pl.ds(tid * RB, RB)])
    plsc.subcore_barrier()
    pltpu.sync_copy(hidx.at[(2 * cid) * NS + tid], idx_v)

    @pl.loop(0, CH16)
    def _(j):
        pltpu.sync_copy(ones_v, acc_a.at[idx_v.at[j]], add=True)

    pltpu.sync_copy(hidx.at[(2 * cid + 1) * NS + tid], idx_v)

    @pl.loop(0, CH16)
    def _(j):
        pltpu.sync_copy(ones_v, acc_b.at[idx_v.at[j]], add=True)

    plsc.subcore_barrier()
    pltpu.sync_copy(acc_a.at[pl.ds(tid * RB, RB)],
                    out.at[pl.ds((2 * cid) * NP + tid * RB, RB)])
    pltpu.sync_copy(acc_b.at[pl.ds(tid * RB, RB)],
                    out.at[pl.ds((2 * cid + 1) * NP + tid * RB, RB)])


_hist = pl.kernel(
    _hist_body,
    out_type=jax.ShapeDtypeStruct((4 * NP, 16), f32),
    mesh=_MESH,
    compiler_params=pltpu.CompilerParams(use_tc_tiling_on_sc=False),
    scratch_types=[
        pltpu.VMEM((CH16, CH), i32),
        pltpu.VMEM((CH, 16), f32),
        pltpu.VMEM((RB, 16), f32),
        pltpu.VMEM_SHARED((NP, 16), f32),
        pltpu.VMEM_SHARED((NP, 16), f32),
    ],
)


def _gs_body(npass, m_flat, isrc, idst, zz, out,
             isrc_v, idst_v, rows_v, z_v, acc):
    cid = lax.axis_index("c")
    tid = lax.axis_index("s")
    pltpu.sync_copy(idst.at[tid], idst_v)
    pltpu.sync_copy(zz, z_v)
    for k in range(npass):
        pltpu.sync_copy(isrc.at[(k * NC + cid) * NS + tid], isrc_v)
        for r in range(RB // ZCH):
            pltpu.sync_copy(z_v, acc.at[pl.ds(tid * RB + r * ZCH, ZCH)])
        plsc.subcore_barrier()

        @pl.loop(0, CH16)
        def _(j):
            pltpu.sync_copy(m_flat.at[isrc_v.at[j]], rows_v)
            pltpu.sync_copy(rows_v, acc.at[idst_v.at[j]], add=True)

        plsc.subcore_barrier()
        pltpu.sync_copy(acc.at[pl.ds(tid * RB, RB)],
                        out.at[pl.ds((k * NC + cid) * NP + tid * RB, RB)])
        if k + 1 < npass:
            plsc.subcore_barrier()


def _mk_gs(npass):
    return pl.kernel(
        functools.partial(_gs_body, npass),
        out_type=jax.ShapeDtypeStruct((npass * NC * NP, QW), f32),
        mesh=_MESH,
        compiler_params=pltpu.CompilerParams(use_tc_tiling_on_sc=False),
        scratch_types=[
            pltpu.VMEM((CH16, CH), i32),
            pltpu.VMEM((CH16, CH), i32),
            pltpu.VMEM((CH, QW), f32),
            pltpu.VMEM((ZCH, QW), f32),
            pltpu.VMEM_SHARED((NP, QW), f32),
        ],
    )


_gs2 = _mk_gs(2)
_gs1 = _mk_gs(1)


def _rs(ref):
    return lax.rsqrt(jnp.maximum(ref[0, :, 0:1], 1.0))


def _quarters(m, n):
    return jnp.stack([m[:, QW * q:QW * (q + 1)] for q in range(n)])


def _tc1_body(x_ref, d_ref, o_ref):
    m = x_ref[...] * _rs(d_ref)
    o_ref[...] = _quarters(m, 4)


def _tc2_body(a_ref, di_ref, do_ref, w_ref, b_ref, o_ref):
    si = _rs(di_ref)
    h = b_ref[...]
    for q in range(4):
        h = h + jnp.dot(a_ref[q] * si, w_ref[QW * q:QW * (q + 1)],
                        preferred_element_type=f32)
    m = jnp.maximum(h, 0.0) * _rs(do_ref)
    o_ref[...] = _quarters(m, 4)


def _tc3_body(a_ref, di_ref, do_ref, w2_ref, b2_ref, w3_ref, o_ref):
    si = _rs(di_ref)
    h = b2_ref[...]
    for q in range(4):
        h = h + jnp.dot(a_ref[q] * si, w2_ref[QW * q:QW * (q + 1)],
                        preferred_element_type=f32)
    h = jnp.maximum(h, 0.0) * _rs(do_ref)
    t = jnp.dot(h, w3_ref[...], preferred_element_type=f32)
    o_ref[...] = _quarters(t, 2)


def _tc4_body(p_ref, di_ref, b_ref, o_ref):
    si = _rs(di_ref)
    o_ref[...] = jnp.concatenate([p_ref[0], p_ref[1]], axis=1) * si + b_ref[...]


def _deg_spec(h):
    return pl.BlockSpec((1, BR, 16), lambda i, h=h: (h, i, 0))


_tc1 = pl.pallas_call(
    _tc1_body,
    grid=(GRID,),
    in_specs=[pl.BlockSpec((BR, DIN), lambda i: (i, 0)), _deg_spec(0)],
    out_specs=pl.BlockSpec((4, BR, QW), lambda i: (0, i, 0)),
    out_shape=jax.ShapeDtypeStruct((4, NN, QW), f32),
)

_tc2 = pl.pallas_call(
    _tc2_body,
    grid=(GRID,),
    in_specs=[
        pl.BlockSpec((4, BR, QW), lambda i: (0, i, 0)),
        _deg_spec(1),
        _deg_spec(2),
        pl.BlockSpec((HID, HID), lambda i: (0, 0)),
        pl.BlockSpec((1, HID), lambda i: (0, 0)),
    ],
    out_specs=pl.BlockSpec((4, BR, QW), lambda i: (0, i, 0)),
    out_shape=jax.ShapeDtypeStruct((4, NN, QW), f32),
)

_tc3 = pl.pallas_call(
    _tc3_body,
    grid=(GRID,),
    in_specs=[
        pl.BlockSpec((4, BR, QW), lambda i: (0, i, 0)),
        _deg_spec(3),
        _deg_spec(0),
        pl.BlockSpec((HID, HID), lambda i: (0, 0)),
        pl.BlockSpec((1, HID), lambda i: (0, 0)),
        pl.BlockSpec((HID, DOUT), lambda i: (0, 0)),
    ],
    out_specs=pl.BlockSpec((2, BR, QW), lambda i: (0, i, 0)),
    out_shape=jax.ShapeDtypeStruct((2, NN, QW), f32),
)

_tc4 = pl.pallas_call(
    _tc4_body,
    grid=(GRID,),
    in_specs=[
        pl.BlockSpec((2, BR, QW), lambda i: (0, i, 0)),
        _deg_spec(1),
        pl.BlockSpec((1, DOUT), lambda i: (0, 0)),
    ],
    out_specs=pl.BlockSpec((BR, DOUT), lambda i: (i, 0)),
    out_shape=jax.ShapeDtypeStruct((NN, DOUT), f32),
)


def kernel(x_driver, x_rider, edge_serves, edge_served_by,
           W1_serves, b1_serves, W1_served_by, b1_served_by,
           W2_serves, b2_serves, W2_served_by, b2_served_by,
           W3_serves, b3_serves, W3_served_by, b3_served_by):
    s_src, s_dst = edge_serves[0], edge_serves[1]
    sb_src, sb_dst = edge_served_by[0], edge_served_by[1]
    pad0 = jnp.zeros((EP - EE,), i32)
    padn = jnp.full((EP - EE,), NN, i32)

    def cat(a, p):
        return jnp.concatenate([a, p])

    hidx = jnp.stack([cat(s_src, padn), cat(s_dst, padn),
                      cat(sb_src, padn), cat(sb_dst, padn)])
    hidx = hidx.reshape(4 * NS, CH16, CH)

    g1s = cat(s_src, pad0)
    g1src = jnp.stack([g1s + q * NN for q in range(4)]).reshape(
        4 * NS, CH16, CH)
    g1dst = cat(s_dst, padn).reshape(NS, CH16, CH)
    g2s = cat(sb_src, pad0)
    g2src = jnp.stack([g2s + q * NN for q in range(4)]).reshape(
        4 * NS, CH16, CH)
    g2dst = cat(sb_dst, padn).reshape(NS, CH16, CH)
    g3src = jnp.stack([g1s, g1s + NN]).reshape(2 * NS, CH16, CH)
    g3dst = g1dst

    ones16 = jnp.ones((CH, 16), f32)
    z16 = jnp.zeros((RB, 16), f32)
    zq = jnp.zeros((ZCH, QW), f32)

    degs = _hist(hidx, ones16, z16).reshape(4, NP, 16)

    m1 = _tc1(x_rider, degs)
    agg1 = _gs2(m1.reshape(4 * NN, QW), g1src, g1dst, zq)
    m2 = _tc2(agg1.reshape(4, NP, QW), degs, degs, W1_serves,
              b1_serves.reshape(1, HID))
    agg2 = _gs2(m2.reshape(4 * NN, QW), g2src, g2dst, zq)
    t3 = _tc3(agg2.reshape(4, NP, QW), degs, degs, W2_served_by,
              b2_served_by.reshape(1, HID), W3_serves)
    p3 = _gs1(t3.reshape(2 * NN, QW), g3src, g3dst, zq)
    return _tc4(p3.reshape(2, NP, QW), degs,
                b3_serves.reshape(1, DOUT))


# speedup vs baseline: 3.8344x; 1.1205x over previous
import functools

import jax
import jax.numpy as jnp
from jax import lax
from jax.experimental import pallas as pl
from jax.experimental.pallas import tpu as pltpu
from jax.experimental.pallas import tpu_sc as plsc

NN = 10000
EE = 160000
DIN = 256
HID = 256
DOUT = 128
QW = 64

NC, NS = 2, 16
CH = 512
EP = 163840
CH16 = EP // (NS * CH)
NP = 10240
RB = NP // NS
ZCH = 128
BR = 400
GRID = NN // BR

f32 = jnp.float32
i32 = jnp.int32

_MESH = plsc.VectorSubcoreMesh(core_axis_name="c", subcore_axis_name="s",
                               num_cores=NC, num_subcores=NS)


def _hist_body(hidx, ones_h, z16, out, idx_v, ones_v, z_v, acc_a, acc_b):
    cid = lax.axis_index("c")
    tid = lax.axis_index("s")
    pltpu.sync_copy(ones_h, ones_v)
    pltpu.sync_copy(z16, z_v)
    pltpu.sync_copy(z_v, acc_a.at[pl.ds(tid * RB, RB)])
    pltpu.sync_copy(z_v, acc_b.at[pl.ds(tid * RB, RB)])
    plsc.subcore_barrier()
    pltpu.sync_copy(hidx.at[(2 * cid) * NS + tid], idx_v)

    @pl.loop(0, CH16)
    def _(j):
        pltpu.sync_copy(ones_v, acc_a.at[idx_v.at[j]], add=True)

    pltpu.sync_copy(hidx.at[(2 * cid + 1) * NS + tid], idx_v)

    @pl.loop(0, CH16)
    def _(j):
        pltpu.sync_copy(ones_v, acc_b.at[idx_v.at[j]], add=True)

    plsc.subcore_barrier()
    pltpu.sync_copy(acc_a.at[pl.ds(tid * RB, RB)],
                    out.at[pl.ds((2 * cid) * NP + tid * RB, RB)])
    pltpu.sync_copy(acc_b.at[pl.ds(tid * RB, RB)],
                    out.at[pl.ds((2 * cid + 1) * NP + tid * RB, RB)])


_hist = pl.kernel(
    _hist_body,
    out_type=jax.ShapeDtypeStruct((4 * NP, 16), f32),
    mesh=_MESH,
    compiler_params=pltpu.CompilerParams(use_tc_tiling_on_sc=False),
    scratch_types=[
        pltpu.VMEM((CH16, CH), i32),
        pltpu.VMEM((CH, 16), f32),
        pltpu.VMEM((RB, 16), f32),
        pltpu.VMEM_SHARED((NP, 16), f32),
        pltpu.VMEM_SHARED((NP, 16), f32),
    ],
)


def _gs_body(npass, m_flat, isrc, idst, zz, out,
             isrc_v, idst_v, rows_v, z_v, acc):
    cid = lax.axis_index("c")
    tid = lax.axis_index("s")
    pltpu.sync_copy(idst.at[tid], idst_v)
    pltpu.sync_copy(zz, z_v)
    for k in range(npass):
        pltpu.sync_copy(isrc.at[(k * NC + cid) * NS + tid], isrc_v)
        for r in range(RB // ZCH):
            pltpu.sync_copy(z_v, acc.at[pl.ds(tid * RB + r * ZCH, ZCH)])
        plsc.subcore_barrier()

        @pl.loop(0, CH16)
        def _(j):
            pltpu.sync_copy(m_flat.at[isrc_v.at[j]], rows_v)
            pltpu.sync_copy(rows_v, acc.at[idst_v.at[j]], add=True)

        plsc.subcore_barrier()
        pltpu.sync_copy(acc.at[pl.ds(tid * RB, RB)],
                        out.at[pl.ds((k * NC + cid) * NP + tid * RB, RB)])
        if k + 1 < npass:
            plsc.subcore_barrier()


def _mk_gs(npass):
    return pl.kernel(
        functools.partial(_gs_body, npass),
        out_type=jax.ShapeDtypeStruct((npass * NC * NP, QW), f32),
        mesh=_MESH,
        compiler_params=pltpu.CompilerParams(use_tc_tiling_on_sc=False),
        scratch_types=[
            pltpu.VMEM((CH16, CH), i32),
            pltpu.VMEM((CH16, CH), i32),
            pltpu.VMEM((CH, QW), f32),
            pltpu.VMEM((ZCH, QW), f32),
            pltpu.VMEM_SHARED((NP, QW), f32),
        ],
    )


_gs2 = _mk_gs(2)
_gs1 = _mk_gs(1)


def _rs(ref):
    return lax.rsqrt(jnp.maximum(ref[0, :, 0:1], 1.0))


def _quarters(m, n):
    return jnp.stack([m[:, QW * q:QW * (q + 1)] for q in range(n)])


def _tc1_body(x_ref, d_ref, o_ref):
    m = x_ref[...] * _rs(d_ref)
    o_ref[...] = _quarters(m, 4)


def _tc2_body(a_ref, di_ref, do_ref, w_ref, b_ref, o_ref):
    si = _rs(di_ref)
    h = b_ref[...]
    for q in range(4):
        h = h + jnp.dot(a_ref[q] * si, w_ref[QW * q:QW * (q + 1)],
                        preferred_element_type=f32)
    m = jnp.maximum(h, 0.0) * _rs(do_ref)
    o_ref[...] = _quarters(m, 4)


def _tc3_body(a_ref, di_ref, do_ref, w2_ref, b2_ref, w3_ref, o_ref):
    si = _rs(di_ref)
    h = b2_ref[...]
    for q in range(4):
        h = h + jnp.dot(a_ref[q] * si, w2_ref[QW * q:QW * (q + 1)],
                        preferred_element_type=f32)
    h = jnp.maximum(h, 0.0) * _rs(do_ref)
    t = jnp.dot(h, w3_ref[...], preferred_element_type=f32)
    o_ref[...] = _quarters(t, 2)


def _tc4_body(p_ref, di_ref, b_ref, o_ref):
    si = _rs(di_ref)
    o_ref[...] = jnp.concatenate([p_ref[0], p_ref[1]], axis=1) * si + b_ref[...]


def _deg_spec(h):
    return pl.BlockSpec((1, BR, 16), lambda i, h=h: (h, i, 0))


_tc1 = pl.pallas_call(
    _tc1_body,
    grid=(GRID,),
    in_specs=[pl.BlockSpec((BR, DIN), lambda i: (i, 0)), _deg_spec(0)],
    out_specs=pl.BlockSpec((4, BR, QW), lambda i: (0, i, 0)),
    out_shape=jax.ShapeDtypeStruct((4, NN, QW), f32),
)

_tc2 = pl.pallas_call(
    _tc2_body,
    grid=(GRID,),
    in_specs=[
        pl.BlockSpec((4, BR, QW), lambda i: (0, i, 0)),
        _deg_spec(1),
        _deg_spec(2),
        pl.BlockSpec((HID, HID), lambda i: (0, 0)),
        pl.BlockSpec((1, HID), lambda i: (0, 0)),
    ],
    out_specs=pl.BlockSpec((4, BR, QW), lambda i: (0, i, 0)),
    out_shape=jax.ShapeDtypeStruct((4, NN, QW), f32),
)

_tc3 = pl.pallas_call(
    _tc3_body,
    grid=(GRID,),
    in_specs=[
        pl.BlockSpec((4, BR, QW), lambda i: (0, i, 0)),
        _deg_spec(3),
        _deg_spec(0),
        pl.BlockSpec((HID, HID), lambda i: (0, 0)),
        pl.BlockSpec((1, HID), lambda i: (0, 0)),
        pl.BlockSpec((HID, DOUT), lambda i: (0, 0)),
    ],
    out_specs=pl.BlockSpec((2, BR, QW), lambda i: (0, i, 0)),
    out_shape=jax.ShapeDtypeStruct((2, NN, QW), f32),
)

_tc4 = pl.pallas_call(
    _tc4_body,
    grid=(GRID,),
    in_specs=[
        pl.BlockSpec((2, BR, QW), lambda i: (0, i, 0)),
        _deg_spec(1),
        pl.BlockSpec((1, DOUT), lambda i: (0, 0)),
    ],
    out_specs=pl.BlockSpec((BR, DOUT), lambda i: (i, 0)),
    out_shape=jax.ShapeDtypeStruct((NN, DOUT), f32),
)


def kernel(x_driver, x_rider, edge_serves, edge_served_by,
           W1_serves, b1_serves, W1_served_by, b1_served_by,
           W2_serves, b2_serves, W2_served_by, b2_served_by,
           W3_serves, b3_serves, W3_served_by, b3_served_by):
    s_src, s_dst = edge_serves[0], edge_serves[1]
    sb_src, sb_dst = edge_served_by[0], edge_served_by[1]
    pad0 = jnp.zeros((EP - EE,), i32)
    padn = jnp.full((EP - EE,), NN, i32)

    def cat(a, p):
        return jnp.concatenate([a, p])

    hidx = jnp.stack([cat(s_src, padn), cat(s_dst, padn),
                      cat(sb_src, padn), cat(sb_dst, padn)])
    hidx = hidx.reshape(4 * NS, CH16, CH)

    g1s = cat(s_src, pad0)
    g1src = jnp.stack([g1s + q * NN for q in range(4)]).reshape(
        4 * NS, CH16, CH)
    g1dst = cat(s_dst, padn).reshape(NS, CH16, CH)
    g2s = cat(sb_src, pad0)
    g2src = jnp.stack([g2s + q * NN for q in range(4)]).reshape(
        4 * NS, CH16, CH)
    g2dst = cat(sb_dst, padn).reshape(NS, CH16, CH)
    g3src = jnp.stack([g1s, g1s + NN]).reshape(2 * NS, CH16, CH)
    g3dst = g1dst

    ones16 = jnp.ones((CH, 16), f32)
    z16 = jnp.zeros((RB, 16), f32)
    zq = jnp.zeros((ZCH, QW), f32)

    degs = _hist(hidx, ones16, z16).reshape(4, NP, 16)

    m1 = _tc1(x_rider, degs)
    agg1 = _gs2(m1.reshape(4 * NN, QW), g1src, g1dst, zq)
    m2 = _tc2(agg1.reshape(4, NP, QW), degs, degs, W1_serves,
              b1_serves.reshape(1, HID))
    agg2 = _gs2(m2.reshape(4 * NN, QW), g2src, g2dst, zq)
    t3 = _tc3(agg2.reshape(4, NP, QW), degs, degs, W2_served_by,
              b2_served_by.reshape(1, HID), W3_serves)
    p3 = _gs1(t3.reshape(2 * NN, QW), g3src, g3dst, zq)
    return _tc4(p3.reshape(2, NP, QW), degs,
                b3_serves.reshape(1, DOUT))
